# baseline (speedup 1.0000x reference)
import jax
import jax.numpy as jnp
from jax.experimental import pallas as pl

_LANES = 128
_BV = 12800


def _contract_kernel(x_ref, e_ref, o_ref):
    bv = x_ref.shape[-1]
    r = bv // _LANES
    x = x_ref[...].reshape(8, r, _LANES)
    e = e_ref[...].reshape(6, r, _LANES)
    a0 = e[0:3]
    a1 = e[3:6]
    x0 = x[0:4]
    x1 = x[4:8]
    y1 = a0[:, None] * x0[None] + a1[:, None] * x1[None]
    y1 = y1.reshape(3, 2, 2, r, _LANES)
    y2 = a0[None, :, None] * y1[:, 0][:, None] \
        + a1[None, :, None] * y1[:, 1][:, None]
    out = a0[None, None] * y2[:, :, 0][:, :, None] \
        + a1[None, None] * y2[:, :, 1][:, :, None]
    o_ref[...] = out.reshape(3, 3, 3, bv)


def kernel(X, e0, e1):
    V = e0.shape[0]
    nblk = -(-V // _BV)
    e6 = jnp.concatenate([e0.T, e1.T], axis=0)
    out = pl.pallas_call(
        _contract_kernel,
        grid=(nblk,),
        in_specs=[
            pl.BlockSpec((2, 2, 2, _BV), lambda i: (0, 0, 0, i)),
            pl.BlockSpec((6, _BV), lambda i: (0, i)),
        ],
        out_specs=pl.BlockSpec((3, 3, 3, _BV), lambda i: (0, 0, 0, i)),
        out_shape=jax.ShapeDtypeStruct((3, 3, 3, V), jnp.float32),
    )(X, e6)
    return out


# speedup vs baseline: 1.2126x; 1.2126x over previous
import jax
import jax.numpy as jnp
from jax.experimental import pallas as pl

_LANES = 128
_BV = 12800


def _contract_kernel(x_ref, a0_ref, a1_ref, o_ref):
    bv = x_ref.shape[-1]
    r = bv // _LANES
    x = x_ref[...].reshape(8, r, _LANES)
    a0 = a0_ref[...].reshape(3, r, _LANES)
    a1 = a1_ref[...].reshape(3, r, _LANES)
    x0 = x[0:4]
    x1 = x[4:8]
    y1 = a0[:, None] * x0[None] + a1[:, None] * x1[None]
    y1 = y1.reshape(3, 2, 2, r, _LANES)
    y2 = a0[None, :, None] * y1[:, 0][:, None] \
        + a1[None, :, None] * y1[:, 1][:, None]
    out = a0[None, None] * y2[:, :, 0][:, :, None] \
        + a1[None, None] * y2[:, :, 1][:, :, None]
    o_ref[...] = out.reshape(3, 3, 3, bv)


def kernel(X, e0, e1):
    V = e0.shape[0]
    nblk = -(-V // _BV)
    out = pl.pallas_call(
        _contract_kernel,
        grid=(nblk,),
        in_specs=[
            pl.BlockSpec((2, 2, 2, _BV), lambda i: (0, 0, 0, i)),
            pl.BlockSpec((3, _BV), lambda i: (0, i)),
            pl.BlockSpec((3, _BV), lambda i: (0, i)),
        ],
        out_specs=pl.BlockSpec((3, 3, 3, _BV), lambda i: (0, 0, 0, i)),
        out_shape=jax.ShapeDtypeStruct((3, 3, 3, V), jnp.float32),
    )(X, e0.T, e1.T)
    return out


# speedup vs baseline: 1.3072x; 1.0780x over previous
import jax
import jax.numpy as jnp
from jax.experimental import pallas as pl

_LANES = 128
_BV = 25600


def _contract_kernel(x_ref, a0_ref, a1_ref, o_ref):
    bv = x_ref.shape[-1]
    r = bv // _LANES
    x = x_ref[...].reshape(8, r, _LANES)
    a0 = a0_ref[...].reshape(3, r, _LANES)
    a1 = a1_ref[...].reshape(3, r, _LANES)
    x0 = x[0:4]
    x1 = x[4:8]
    y1 = a0[:, None] * x0[None] + a1[:, None] * x1[None]
    y1 = y1.reshape(3, 2, 2, r, _LANES)
    y2 = a0[None, :, None] * y1[:, 0][:, None] \
        + a1[None, :, None] * y1[:, 1][:, None]
    out = a0[None, None] * y2[:, :, 0][:, :, None] \
        + a1[None, None] * y2[:, :, 1][:, :, None]
    o_ref[...] = out.reshape(3, 3, 3, bv)


def kernel(X, e0, e1):
    V = e0.shape[0]
    nblk = -(-V // _BV)
    out = pl.pallas_call(
        _contract_kernel,
        grid=(nblk,),
        in_specs=[
            pl.BlockSpec((2, 2, 2, _BV), lambda i: (0, 0, 0, i)),
            pl.BlockSpec((3, _BV), lambda i: (0, i)),
            pl.BlockSpec((3, _BV), lambda i: (0, i)),
        ],
        out_specs=pl.BlockSpec((3, 3, 3, _BV), lambda i: (0, 0, 0, i)),
        out_shape=jax.ShapeDtypeStruct((3, 3, 3, V), jnp.float32),
    )(X, e0.T, e1.T)
    return out
